# Initial kernel scaffold; baseline (speedup 1.0000x reference)
import functools

import jax
import jax.numpy as jnp
from jax import lax
from jax.experimental import pallas as pl
from jax.experimental.pallas import tpu as pltpu
from jax.experimental.pallas import tpu_sc as plsc

N = 10000
E = 320000
D = 128
L = 2

NC = 2
NS = 16
LANES = 16
NW = NC * NS

CH = 128
NCHUNK = -(-E // (NW * CH))
EPT = NCHUNK * CH
E_PAD = EPT * NW
RPT = N // NS

_mesh = plsc.VectorSubcoreMesh(core_axis_name="c", subcore_axis_name="s")


@functools.partial(
    pl.kernel,
    out_type=jax.ShapeDtypeStruct((NC, N, D), jnp.float32),
    mesh=_mesh,
    scratch_types=[
        pltpu.VMEM((NCHUNK, CH), jnp.int32),
        pltpu.VMEM((NCHUNK, CH), jnp.int32),
        pltpu.VMEM((EPT,), jnp.float32),
        pltpu.VMEM((CH, D), jnp.float32),
        pltpu.VMEM_SHARED((N, D), jnp.float32),
        pltpu.SemaphoreType.DMA,
    ],
)
def _spmm_sc(h_hbm, cols_hbm, rows_hbm, lap_hbm, zeros_hbm, out_hbm,
             colv, rowv, lapv, gbuf, acc, sem):
    c = lax.axis_index("c")
    s = lax.axis_index("s")
    wid = c * NS + s

    pltpu.sync_copy(zeros_hbm, acc.at[pl.ds(s * RPT, RPT)])

    pltpu.sync_copy(cols_hbm.at[wid], colv)
    pltpu.sync_copy(rows_hbm.at[wid], rowv)
    pltpu.sync_copy(lap_hbm.at[wid], lapv)

    plsc.subcore_barrier()

    def chunk_body(j, carry):
        pltpu.async_copy(h_hbm.at[colv.at[j]], gbuf, sem).wait()

        def edge_body(k, carry2):
            lv = plsc.load_gather(
                lapv, [jnp.full((LANES,), j * CH + k, jnp.int32)])
            for g in range(D // LANES):
                sl = pl.ds(g * LANES, LANES)
                gbuf[k, sl] = gbuf[k, sl] * lv
            return carry2

        lax.fori_loop(0, CH, edge_body, 0, unroll=2)

        pltpu.sync_copy(gbuf, acc.at[rowv.at[j]], add=True)
        return carry

    lax.fori_loop(0, NCHUNK, chunk_body, 0)

    plsc.subcore_barrier()

    pltpu.sync_copy(acc.at[pl.ds(s * RPT, RPT)],
                    out_hbm.at[c, pl.ds(s * RPT, RPT)])


def _dense_body(h_ref, p0_ref, p1_ref, wt_ref, b_ref, o_ref):
    x = h_ref[...] - (p0_ref[...] + p1_ref[...])
    y = jnp.dot(x, wt_ref[...], preferred_element_type=jnp.float32)
    o_ref[...] = jnp.maximum(y + b_ref[...], 0.0)


_BN = 1000


def _dense(h, p0, p1, wt, b2):
    grid = (N // _BN,)
    return pl.pallas_call(
        _dense_body,
        grid=grid,
        in_specs=[
            pl.BlockSpec((_BN, D), lambda i: (i, 0)),
            pl.BlockSpec((_BN, D), lambda i: (i, 0)),
            pl.BlockSpec((_BN, D), lambda i: (i, 0)),
            pl.BlockSpec((D, D), lambda i: (0, 0)),
            pl.BlockSpec((1, D), lambda i: (0, 0)),
        ],
        out_specs=pl.BlockSpec((_BN, D), lambda i: (i, 0)),
        out_shape=jax.ShapeDtypeStruct((N, D), jnp.float32),
    )(h, p0, p1, wt, b2)


def kernel(edge_index, lap_values, X, W, b, temp_global):
    rows = edge_index[0]
    cols = edge_index[1]
    pad = E_PAD - E
    cols_p = jnp.pad(cols, (0, pad)).reshape(NW, NCHUNK, CH)
    rows_p = jnp.pad(rows, (0, pad)).reshape(NW, NCHUNK, CH)
    lap_p = jnp.pad(lap_values, (0, pad))
    zeros = jnp.zeros((RPT, D), jnp.float32)

    h = X
    for i in range(L):
        lap_i = (lap_p * temp_global[i]).reshape(NW, EPT)
        P = _spmm_sc(h, cols_p, rows_p, lap_i, zeros)
        h = _dense(h, P[0], P[1], W[i].T, b[i].reshape(1, D))
    return h


# speedup vs baseline: 4.0525x; 4.0525x over previous
import functools

import jax
import jax.numpy as jnp
from jax import lax
from jax.experimental import pallas as pl
from jax.experimental.pallas import tpu as pltpu
from jax.experimental.pallas import tpu_sc as plsc

N = 10000
E = 320000
D = 128
L = 2

NC = 2
NS = 16
LANES = 16
NW = NC * NS

CH = 128
NCHUNK = -(-E // (NW * CH))
EPT = NCHUNK * CH
E_PAD = EPT * NW
N_PAD = 10240
RPT = N_PAD // NS

_mesh = plsc.VectorSubcoreMesh(core_axis_name="c", subcore_axis_name="s")


@functools.partial(
    pl.kernel,
    out_type=jax.ShapeDtypeStruct((NC, N_PAD, D), jnp.float32),
    mesh=_mesh,
    scratch_types=[
        pltpu.VMEM((NCHUNK, CH), jnp.int32),
        pltpu.VMEM((NCHUNK, CH), jnp.int32),
        pltpu.VMEM((EPT + LANES,), jnp.float32),
        pltpu.VMEM((CH, D), jnp.float32),
        pltpu.VMEM_SHARED((N_PAD, D), jnp.float32),
        pltpu.SemaphoreType.DMA,
    ],
)
def _spmm_sc(h_hbm, cols_hbm, rows_hbm, lap_hbm, zeros_hbm, out_hbm,
             colv, rowv, lapv, gbuf, acc, sem):
    c = lax.axis_index("c")
    s = lax.axis_index("s")
    wid = c * NS + s

    pltpu.sync_copy(zeros_hbm, acc.at[pl.ds(s * RPT, RPT)])

    pltpu.sync_copy(cols_hbm.at[wid], colv)
    pltpu.sync_copy(rows_hbm.at[wid], rowv)
    pltpu.sync_copy(lap_hbm.at[pl.ds(wid * EPT, EPT)], lapv.at[pl.ds(0, EPT)])

    plsc.subcore_barrier()

    def chunk_body(j, carry):
        pltpu.async_copy(h_hbm.at[colv.at[j]], gbuf, sem).wait()

        def edge_body(k, carry2):
            lv = lapv[pl.ds(j * CH + k, LANES)][0]
            for g in range(D // LANES):
                sl = pl.ds(g * LANES, LANES)
                gbuf[k, sl] = gbuf[k, sl] * lv
            return carry2

        lax.fori_loop(0, CH, edge_body, 0, unroll=2)

        pltpu.sync_copy(gbuf, acc.at[rowv.at[j]], add=True)
        return carry

    lax.fori_loop(0, NCHUNK, chunk_body, 0)

    plsc.subcore_barrier()

    pltpu.sync_copy(acc.at[pl.ds(s * RPT, RPT)],
                    out_hbm.at[c, pl.ds(s * RPT, RPT)])


def _dense_body(h_ref, p_ref, wt_ref, b_ref, o_ref):
    x = h_ref[...] - (p_ref[0] + p_ref[1])
    y = jnp.dot(x, wt_ref[...], preferred_element_type=jnp.float32)
    o_ref[...] = jnp.maximum(y + b_ref[...], 0.0)


_BN = 1000


def _dense(h, P, wt, b2):
    grid = (N // _BN,)
    return pl.pallas_call(
        _dense_body,
        grid=grid,
        in_specs=[
            pl.BlockSpec((_BN, D), lambda i: (i, 0)),
            pl.BlockSpec((NC, _BN, D), lambda i: (0, i, 0)),
            pl.BlockSpec((D, D), lambda i: (0, 0)),
            pl.BlockSpec((1, D), lambda i: (0, 0)),
        ],
        out_specs=pl.BlockSpec((_BN, D), lambda i: (i, 0)),
        out_shape=jax.ShapeDtypeStruct((N, D), jnp.float32),
    )(h, P, wt, b2)


def kernel(edge_index, lap_values, X, W, b, temp_global):
    rows = edge_index[0]
    cols = edge_index[1]
    pad = E_PAD - E
    cols_p = jnp.pad(cols, (0, pad)).reshape(NW, NCHUNK, CH)
    rows_p = jnp.pad(rows, (0, pad)).reshape(NW, NCHUNK, CH)
    lap_p = jnp.pad(lap_values, (0, pad))
    zeros = jnp.zeros((RPT, D), jnp.float32)

    h = X
    for i in range(L):
        lap_i = lap_p * temp_global[i]
        P = _spmm_sc(h, cols_p, rows_p, lap_i, zeros)
        h = _dense(h, P, W[i].T, b[i].reshape(1, D))
    return h
